# baseline (speedup 1.0000x reference)
import functools

import jax
import jax.numpy as jnp
from jax import lax
from jax.experimental import pallas as pl
from jax.experimental.pallas import tpu as pltpu
from jax.experimental.pallas import tpu_sc as plsc

_NC = 2
_NS = 16
_NW = _NC * _NS


def _tc_pre_body(x_ref, wa_t_ref, wb_t_ref, ta_ref, tb_ref):
    xv = x_ref[...]
    ta_ref[...] = jnp.dot(xv, wa_t_ref[...], preferred_element_type=jnp.float32)
    tb_ref[...] = jnp.dot(xv, wb_t_ref[...], preferred_element_type=jnp.float32)


def _tc_attr_body(ea_ref, wc_t_ref, be1_ref, a_ref):
    a_ref[...] = (
        jnp.dot(ea_ref[...], wc_t_ref[...], preferred_element_type=jnp.float32)
        + be1_ref[...]
    )


def _make_sc_edge_kernel(N, E, D, C):
    EW = E // _NW
    NCH = EW // C
    NB = NCH // 2
    NPAD = -(-N // (_NS * 8)) * (_NS * 8)
    RPT = NPAD // _NS
    KD = D // 16
    DOTR = -(-C // 16) * 16

    mesh = plsc.VectorSubcoreMesh(core_axis_name="c", subcore_axis_name="s")

    @functools.partial(
        pl.kernel,
        out_type=[
            jax.ShapeDtypeStruct((_NC, NPAD, D), jnp.float32),
            jax.ShapeDtypeStruct((_NC, NPAD, 16), jnp.float32),
        ],
        mesh=mesh,
        compiler_params=pltpu.CompilerParams(
            needs_layout_passes=False, use_tc_tiling_on_sc=False
        ),
        scratch_types=[
            pltpu.VMEM((2, 2, 2, C), jnp.int32),
            pltpu.VMEM((C, D), jnp.float32),
            pltpu.VMEM((C, D), jnp.float32),
            pltpu.VMEM((C, D), jnp.float32),
            pltpu.VMEM((C, D), jnp.float32),
            pltpu.VMEM((C, D), jnp.float32),
            pltpu.VMEM((C, D), jnp.float32),
            pltpu.VMEM((C, D), jnp.float32),
            pltpu.VMEM((DOTR, 16), jnp.float32),
            pltpu.VMEM((DOTR, 16), jnp.float32),
            pltpu.VMEM((DOTR, 16), jnp.float32),
            pltpu.VMEM((D,), jnp.float32),
            pltpu.VMEM((16,), jnp.float32),
            pltpu.VMEM_SHARED((NPAD, D), jnp.float32),
            pltpu.VMEM_SHARED((NPAD, 16), jnp.float32),
            pltpu.SemaphoreType.DMA,
            pltpu.SemaphoreType.DMA,
            pltpu.SemaphoreType.DMA,
        ],
    )
    def sc_edge(ta_h, tb_h, a_h, idx_h, u_h, cst_h,
                hp_h, sp_h,
                islot, buf_a0, buf_a1, buf_b0, buf_b1, buf_c0, buf_c1,
                h_buf, cpr, cpc, dot_buf, u_buf, cst_buf,
                h_acc, s_acc, sem_g, sem_s, sem_i):
        cid = lax.axis_index("c")
        sid = lax.axis_index("s")
        wid = sid * _NC + cid
        buf_a = (buf_a0, buf_a1)
        buf_b = (buf_b0, buf_b1)
        buf_c = (buf_c0, buf_c1)

        pltpu.sync_copy(u_h, u_buf)
        pltpu.sync_copy(cst_h, cst_buf)

        zero16 = jnp.zeros((16,), jnp.float32)

        def zero_h_row(j, carry):
            for k in range(KD):
                h_buf[j, pl.ds(k * 16, 16)] = zero16
            return carry

        lax.fori_loop(0, C, zero_h_row, 0)

        def zero_c_row(j, carry):
            cpc[j, :] = zero16
            return carry

        lax.fori_loop(0, DOTR, zero_c_row, 0)

        base_r = sid * RPT
        nzf, rem = RPT // C, RPT % C
        for t in range(nzf):
            pltpu.sync_copy(h_buf, h_acc.at[pl.ds(base_r + t * C, C)])
        if rem:
            pltpu.sync_copy(h_buf.at[pl.ds(0, rem)],
                            h_acc.at[pl.ds(base_r + nzf * C, rem)])
        nsf, srem = RPT // DOTR, RPT % DOTR
        for t in range(nsf):
            pltpu.sync_copy(cpc, s_acc.at[pl.ds(base_r + t * DOTR, DOTR)])
        if srem:
            pltpu.sync_copy(cpc.at[pl.ds(0, srem)],
                            s_acc.at[pl.ds(base_r + nsf * DOTR, srem)])
        plsc.subcore_barrier()

        cst_v = cst_buf[:]
        k1 = cst_v[0]
        k2 = cst_v[1]
        k3 = cst_v[2]
        iota16 = lax.iota(jnp.int32, 16)
        zero_i16 = jnp.zeros((16,), jnp.int32)
        u_regs = [u_buf[pl.ds(k * 16, 16)] for k in range(KD)]

        onehot1 = jnp.where(iota16 == 1, 1.0, 0.0).astype(jnp.float32)

        def init_c_row(e, carry):
            cpr[e, :] = onehot1
            return carry

        lax.fori_loop(0, DOTR, init_c_row, 0)

        ew_base = wid * EW

        def issue_idx(o_next, po_next):
            pltpu.async_copy(idx_h.at[wid, pl.ds(2 * o_next, 2)],
                             islot.at[po_next], sem_i)

        def drain_idx(o_next, po_next):
            pltpu.make_async_copy(idx_h.at[wid, pl.ds(2 * o_next, 2)],
                                  islot.at[po_next], sem_i).wait()

        def issue_gathers(t, po, s):
            base = ew_base + t * C
            pltpu.async_copy(ta_h.at[islot.at[po, s, 0]], buf_a[s], sem_g)
            pltpu.async_copy(tb_h.at[islot.at[po, s, 1]], buf_b[s], sem_g)
            pltpu.async_copy(a_h.at[pl.ds(base, C), :], buf_c[s], sem_g)

        def drain_gathers(t, po, s):
            base = ew_base + t * C
            pltpu.make_async_copy(ta_h.at[islot.at[po, s, 0]], buf_a[s],
                                  sem_g).wait()
            pltpu.make_async_copy(tb_h.at[islot.at[po, s, 1]], buf_b[s],
                                  sem_g).wait()
            pltpu.make_async_copy(a_h.at[pl.ds(base, C), :], buf_c[s],
                                  sem_g).wait()

        def issue_scatters(po, s):
            pltpu.async_copy(h_buf, h_acc.at[islot.at[po, s, 0]], sem_s,
                             add=True)
            pltpu.async_copy(cpr.at[pl.ds(0, C)], s_acc.at[islot.at[po, s, 0]],
                             sem_s, add=True)
            pltpu.async_copy(cpc.at[pl.ds(0, C)], s_acc.at[islot.at[po, s, 1]],
                             sem_s, add=True)

        def drain_scatters(po, s):
            pltpu.make_async_copy(h_buf, h_acc.at[islot.at[po, s, 0]],
                                  sem_s).wait()
            pltpu.make_async_copy(cpr.at[pl.ds(0, C)],
                                  s_acc.at[islot.at[po, s, 0]], sem_s).wait()
            pltpu.make_async_copy(cpc.at[pl.ds(0, C)],
                                  s_acc.at[islot.at[po, s, 1]], sem_s).wait()

        def compute_chunk(s):
            ba, bb, bc = buf_a[s], buf_b[s], buf_c[s]

            @plsc.parallel_loop(0, C, 1, unroll=4)
            def edge_body(e):
                acc = zero16
                for k in range(KD):
                    hk = jnp.maximum(
                        ba[e, pl.ds(k * 16, 16)]
                        + bb[e, pl.ds(k * 16, 16)]
                        + bc[e, pl.ds(k * 16, 16)],
                        0.0,
                    )
                    h_buf[e, pl.ds(k * 16, 16)] = hk
                    acc = acc + hk * u_regs[k]
                dot_buf[e, :] = acc

            @plsc.parallel_loop(0, DOTR // 16, 1)
            def c_body(g):
                erows = g * 16 + iota16
                cols = [plsc.load_gather(dot_buf, [erows, zero_i16 + j])
                        for j in range(16)]
                while len(cols) > 1:
                    cols = [cols[i] + cols[i + 1]
                            for i in range(0, len(cols), 2)]
                c16 = jnp.maximum(cols[0] + k1, 0.0) * k2 + k3
                plsc.store_scatter(cpr, [erows, zero_i16], c16)
                plsc.store_scatter(cpc, [erows, zero_i16 + 2], c16)

        pltpu.sync_copy(idx_h.at[wid, pl.ds(0, 2)], islot.at[0])
        issue_gathers(0, 0, 0)

        def outer_body(o, carry):
            po = jnp.bitwise_and(o, 1)
            po1 = 1 - po
            t0 = 2 * o
            @pl.when(o > 0)
            def _():
                drain_scatters(po1, 1)
            @pl.when(o < NB - 1)
            def _():
                issue_idx(o + 1, po1)
            issue_gathers(t0 + 1, po, 1)
            drain_gathers(t0, po, 0)
            compute_chunk(0)
            issue_scatters(po, 0)
            drain_scatters(po, 0)
            @pl.when(o < NB - 1)
            def _():
                drain_idx(o + 1, po1)
                issue_gathers(t0 + 2, po1, 0)
            drain_gathers(t0 + 1, po, 1)
            compute_chunk(1)
            issue_scatters(po, 1)
            return carry

        lax.fori_loop(0, NB, outer_body, 0)
        drain_scatters((NB - 1) % 2, 1)

        plsc.subcore_barrier()
        sl = pl.ds(base_r, RPT)
        pltpu.sync_copy(h_acc.at[sl], hp_h.at[cid, sl])
        pltpu.sync_copy(s_acc.at[sl], sp_h.at[cid, sl])

    return sc_edge


def _tc_post_body(x_ref, hp_ref, sp_ref, pos_ref,
                  we2_t_ref, be2_ref, wn1a_t_ref, wn1b_t_ref, bn1_ref,
                  wn2_t_ref, bn2_ref, nf_ref, pos_out_ref):
    n = x_ref.shape[0]
    H = hp_ref[0, :n] + hp_ref[1, :n]
    S = sp_ref[0, :n] + sp_ref[1, :n]
    cnt = S[:, 1:2]
    agg = (
        jnp.dot(H, we2_t_ref[...], preferred_element_type=jnp.float32)
        + cnt * be2_ref[...]
    )
    hn = jnp.maximum(
        jnp.dot(x_ref[...], wn1a_t_ref[...], preferred_element_type=jnp.float32)
        + jnp.dot(agg, wn1b_t_ref[...], preferred_element_type=jnp.float32)
        + bn1_ref[...],
        0.0,
    )
    nf_ref[...] = (
        jnp.dot(hn, wn2_t_ref[...], preferred_element_type=jnp.float32)
        + bn2_ref[...]
    )
    s = S[:, 0:1] - S[:, 2:3]
    coord = jnp.sum(s * pos_ref[...], axis=0, keepdims=True)
    pos_out_ref[...] = pos_ref[...] + coord


def kernel(x, edge_index, edge_attr, pos,
           We1, be1, We2, be2, Wn1, bn1, Wn2, bn2, Wc1, bc1, Wc2, bc2):
    N, D = x.shape
    E = edge_index.shape[1]
    C = 40
    NCH = E // _NW // C

    wa_t = We1[:, :D].T
    wb_t = We1[:, D:2 * D].T
    wc_t = We1[:, 2 * D:].T
    u = We2.T @ Wc1[0]
    k1 = be2 @ Wc1[0] + bc1[0]
    consts = jnp.zeros((16,), jnp.float32)
    consts = consts.at[0].set(k1).at[1].set(Wc2[0, 0]).at[2].set(bc2[0])
    idx4 = jnp.stack(
        [edge_index[0].reshape(_NW, NCH, C), edge_index[1].reshape(_NW, NCH, C)],
        axis=2,
    )

    ta, tb = pl.pallas_call(
        _tc_pre_body,
        out_shape=[
            jax.ShapeDtypeStruct((N, D), jnp.float32),
            jax.ShapeDtypeStruct((N, D), jnp.float32),
        ],
    )(x, wa_t, wb_t)

    BE = 4000
    a_tab = pl.pallas_call(
        _tc_attr_body,
        grid=(E // BE,),
        in_specs=[
            pl.BlockSpec((BE, edge_attr.shape[1]), lambda i: (i, 0)),
            pl.BlockSpec(wc_t.shape, lambda i: (0, 0)),
            pl.BlockSpec((1, D), lambda i: (0, 0)),
        ],
        out_specs=pl.BlockSpec((BE, D), lambda i: (i, 0)),
        out_shape=jax.ShapeDtypeStruct((E, D), jnp.float32),
    )(edge_attr, wc_t, be1[None, :])

    hp, sp = _make_sc_edge_kernel(N, E, D, C)(ta, tb, a_tab, idx4, u, consts)

    nf, pos_out = pl.pallas_call(
        _tc_post_body,
        out_shape=[
            jax.ShapeDtypeStruct((N, D), jnp.float32),
            jax.ShapeDtypeStruct(pos.shape, jnp.float32),
        ],
    )(x, hp, sp, pos,
      We2.T, be2[None, :], Wn1[:, :D].T, Wn1[:, D:].T, bn1[None, :],
      Wn2.T, bn2[None, :])
    return (nf, pos_out)


# speedup vs baseline: 1.0530x; 1.0530x over previous
import functools

import numpy as np

import jax
import jax.numpy as jnp
from jax import lax
from jax.experimental import pallas as pl
from jax.experimental.pallas import tpu as pltpu
from jax.experimental.pallas import tpu_sc as plsc

_NC = 2
_NS = 16
_NW = _NC * _NS


def _tc_pre_body(x_ref, wa_t_ref, wb_t_ref, ta_ref, tb_ref):
    xv = x_ref[...]
    ta_ref[...] = jnp.dot(
        xv, wa_t_ref[...], preferred_element_type=jnp.float32
    ).astype(jnp.bfloat16)
    tb_ref[...] = jnp.dot(
        xv, wb_t_ref[...], preferred_element_type=jnp.float32
    ).astype(jnp.bfloat16)


def _tc_attr_body(ea_ref, wc_t_ref, be1_ref, a_ref):
    a_ref[...] = (
        jnp.dot(ea_ref[...], wc_t_ref[...], preferred_element_type=jnp.float32)
        + be1_ref[...]
    )


def _make_sc_edge_kernel(N, E, D, C):
    EW = E // _NW
    NCH = EW // C
    NB = NCH // 2
    NPAD = -(-N // (_NS * 8)) * (_NS * 8)
    RPT = NPAD // _NS
    KD = D // 16
    DOTR = -(-C // 16) * 16

    mesh = plsc.VectorSubcoreMesh(core_axis_name="c", subcore_axis_name="s")

    @functools.partial(
        pl.kernel,
        out_type=[
            jax.ShapeDtypeStruct((_NC, NPAD, D), jnp.float32),
            jax.ShapeDtypeStruct((_NC, NPAD, 16), jnp.float32),
        ],
        mesh=mesh,
        compiler_params=pltpu.CompilerParams(
            needs_layout_passes=False, use_tc_tiling_on_sc=False
        ),
        scratch_types=[
            pltpu.VMEM((2, 2, 2, C), jnp.int32),
            pltpu.VMEM((C, D), jnp.bfloat16),
            pltpu.VMEM((C, D), jnp.bfloat16),
            pltpu.VMEM((C, D), jnp.bfloat16),
            pltpu.VMEM((C, D), jnp.bfloat16),
            pltpu.VMEM((C, D), jnp.float32),
            pltpu.VMEM((C, D), jnp.float32),
            pltpu.VMEM((C, D), jnp.float32),
            pltpu.VMEM((DOTR, 16), jnp.float32),
            pltpu.VMEM((DOTR, 16), jnp.float32),
            pltpu.VMEM((DOTR, 16), jnp.float32),
            pltpu.VMEM((D,), jnp.float32),
            pltpu.VMEM((16,), jnp.float32),
            pltpu.VMEM_SHARED((NPAD, D), jnp.float32),
            pltpu.VMEM_SHARED((NPAD, 16), jnp.float32),
            pltpu.SemaphoreType.DMA,
            pltpu.SemaphoreType.DMA,
            pltpu.SemaphoreType.DMA,
        ],
    )
    def sc_edge(ta_h, tb_h, a_h, idx_h, u_h, cst_h,
                hp_h, sp_h,
                islot, buf_a0, buf_a1, buf_b0, buf_b1, buf_c0, buf_c1,
                h_buf, cpr, cpc, dot_buf, u_buf, cst_buf,
                h_acc, s_acc, sem_g, sem_s, sem_i):
        cid = lax.axis_index("c")
        sid = lax.axis_index("s")
        wid = sid * _NC + cid
        buf_a = (buf_a0, buf_a1)
        buf_b = (buf_b0, buf_b1)
        buf_c = (buf_c0, buf_c1)

        pltpu.sync_copy(u_h, u_buf)
        pltpu.sync_copy(cst_h, cst_buf)

        zero16 = jnp.zeros((16,), jnp.float32)

        def zero_h_row(j, carry):
            for k in range(KD):
                h_buf[j, pl.ds(k * 16, 16)] = zero16
            return carry

        lax.fori_loop(0, C, zero_h_row, 0)

        def zero_c_row(j, carry):
            cpc[j, :] = zero16
            return carry

        lax.fori_loop(0, DOTR, zero_c_row, 0)

        base_r = sid * RPT
        nzf, rem = RPT // C, RPT % C
        for t in range(nzf):
            pltpu.sync_copy(h_buf, h_acc.at[pl.ds(base_r + t * C, C)])
        if rem:
            pltpu.sync_copy(h_buf.at[pl.ds(0, rem)],
                            h_acc.at[pl.ds(base_r + nzf * C, rem)])
        nsf, srem = RPT // DOTR, RPT % DOTR
        for t in range(nsf):
            pltpu.sync_copy(cpc, s_acc.at[pl.ds(base_r + t * DOTR, DOTR)])
        if srem:
            pltpu.sync_copy(cpc.at[pl.ds(0, srem)],
                            s_acc.at[pl.ds(base_r + nsf * DOTR, srem)])
        plsc.subcore_barrier()

        cst_v = cst_buf[:]
        k1 = cst_v[0]
        k2 = cst_v[1]
        k3 = cst_v[2]
        iota16 = lax.iota(jnp.int32, 16)
        zero_i16 = jnp.zeros((16,), jnp.int32)
        u_regs = [u_buf[pl.ds(k * 16, 16)] for k in range(KD)]

        onehot1 = jnp.where(iota16 == 1, 1.0, 0.0).astype(jnp.float32)

        def init_c_row(e, carry):
            cpr[e, :] = onehot1
            return carry

        lax.fori_loop(0, DOTR, init_c_row, 0)

        ew_base = wid * EW

        def issue_idx(o_next, po_next):
            pltpu.async_copy(idx_h.at[wid, pl.ds(2 * o_next, 2)],
                             islot.at[po_next], sem_i)

        def drain_idx(o_next, po_next):
            pltpu.make_async_copy(idx_h.at[wid, pl.ds(2 * o_next, 2)],
                                  islot.at[po_next], sem_i).wait()

        def issue_gathers(t, po, s):
            base = ew_base + t * C
            pltpu.async_copy(ta_h.at[islot.at[po, s, 0]], buf_a[s], sem_g)
            pltpu.async_copy(tb_h.at[islot.at[po, s, 1]], buf_b[s], sem_g)
            pltpu.async_copy(a_h.at[pl.ds(base, C), :], buf_c[s], sem_g)

        def drain_gathers(t, po, s):
            base = ew_base + t * C
            pltpu.make_async_copy(ta_h.at[islot.at[po, s, 0]], buf_a[s],
                                  sem_g).wait()
            pltpu.make_async_copy(tb_h.at[islot.at[po, s, 1]], buf_b[s],
                                  sem_g).wait()
            pltpu.make_async_copy(a_h.at[pl.ds(base, C), :], buf_c[s],
                                  sem_g).wait()

        def issue_scatters(po, s):
            pltpu.async_copy(h_buf, h_acc.at[islot.at[po, s, 0]], sem_s,
                             add=True)
            pltpu.async_copy(cpr.at[pl.ds(0, C)], s_acc.at[islot.at[po, s, 0]],
                             sem_s, add=True)
            pltpu.async_copy(cpc.at[pl.ds(0, C)], s_acc.at[islot.at[po, s, 1]],
                             sem_s, add=True)

        def drain_scatters(po, s):
            pltpu.make_async_copy(h_buf, h_acc.at[islot.at[po, s, 0]],
                                  sem_s).wait()
            pltpu.make_async_copy(cpr.at[pl.ds(0, C)],
                                  s_acc.at[islot.at[po, s, 0]], sem_s).wait()
            pltpu.make_async_copy(cpc.at[pl.ds(0, C)],
                                  s_acc.at[islot.at[po, s, 1]], sem_s).wait()

        def compute_chunk(s):
            ba, bb, bc = buf_a[s], buf_b[s], buf_c[s]

            @plsc.parallel_loop(0, C, 1, unroll=2)
            def edge_body(e):
                acc = zero16
                for k in range(KD // 2):
                    va = plsc.unpack(ba[e, pl.ds(k * 32, 32)],
                                     format=plsc.PackFormat.INTERLEAVED)
                    vb = plsc.unpack(bb[e, pl.ds(k * 32, 32)],
                                     format=plsc.PackFormat.INTERLEAVED)
                    for j in (0, 1):
                        m = 2 * k + j
                        hk = jnp.maximum(
                            va[j] + vb[j] + bc[e, pl.ds(m * 16, 16)], 0.0
                        )
                        h_buf[e, pl.ds(m * 16, 16)] = hk
                        acc = acc + hk * u_regs[m]
                dot_buf[e, :] = acc

            @plsc.parallel_loop(0, DOTR // 16, 1)
            def c_body(g):
                erows = g * 16 + iota16
                cols = [plsc.load_gather(dot_buf, [erows, zero_i16 + j])
                        for j in range(16)]
                while len(cols) > 1:
                    cols = [cols[i] + cols[i + 1]
                            for i in range(0, len(cols), 2)]
                c16 = jnp.maximum(cols[0] + k1, 0.0) * k2 + k3
                plsc.store_scatter(cpr, [erows, zero_i16], c16)
                plsc.store_scatter(cpc, [erows, zero_i16 + 2], c16)

        pltpu.sync_copy(idx_h.at[wid, pl.ds(0, 2)], islot.at[0])
        issue_gathers(0, 0, 0)

        def outer_body(o, carry):
            po = jnp.bitwise_and(o, 1)
            po1 = 1 - po
            t0 = 2 * o
            @pl.when(o > 0)
            def _():
                drain_scatters(po1, 1)
            @pl.when(o < NB - 1)
            def _():
                issue_idx(o + 1, po1)
            issue_gathers(t0 + 1, po, 1)
            drain_gathers(t0, po, 0)
            compute_chunk(0)
            issue_scatters(po, 0)
            drain_scatters(po, 0)
            @pl.when(o < NB - 1)
            def _():
                drain_idx(o + 1, po1)
                issue_gathers(t0 + 2, po1, 0)
            drain_gathers(t0 + 1, po, 1)
            compute_chunk(1)
            issue_scatters(po, 1)
            return carry

        lax.fori_loop(0, NB, outer_body, 0)
        drain_scatters((NB - 1) % 2, 1)

        plsc.subcore_barrier()
        sl = pl.ds(base_r, RPT)
        pltpu.sync_copy(h_acc.at[sl], hp_h.at[cid, sl])
        pltpu.sync_copy(s_acc.at[sl], sp_h.at[cid, sl])

    return sc_edge


def _tc_post_body(x_ref, hp_ref, sp_ref, pos_ref,
                  we2_t_ref, be2_ref, wn1a_t_ref, wn1b_t_ref, bn1_ref,
                  wn2_t_ref, bn2_ref, nf_ref, pos_out_ref):
    n = x_ref.shape[0]
    H = hp_ref[0, :n] + hp_ref[1, :n]
    S = sp_ref[0, :n] + sp_ref[1, :n]
    cnt = S[:, 1:2]
    agg = (
        jnp.dot(H, we2_t_ref[...], preferred_element_type=jnp.float32)
        + cnt * be2_ref[...]
    )
    hn = jnp.maximum(
        jnp.dot(x_ref[...], wn1a_t_ref[...], preferred_element_type=jnp.float32)
        + jnp.dot(agg, wn1b_t_ref[...], preferred_element_type=jnp.float32)
        + bn1_ref[...],
        0.0,
    )
    nf_ref[...] = (
        jnp.dot(hn, wn2_t_ref[...], preferred_element_type=jnp.float32)
        + bn2_ref[...]
    )
    s = S[:, 0:1] - S[:, 2:3]
    coord = jnp.sum(s * pos_ref[...], axis=0, keepdims=True)
    pos_out_ref[...] = pos_ref[...] + coord


def kernel(x, edge_index, edge_attr, pos,
           We1, be1, We2, be2, Wn1, bn1, Wn2, bn2, Wc1, bc1, Wc2, bc2):
    N, D = x.shape
    E = edge_index.shape[1]
    C = 40
    NCH = E // _NW // C

    pi = np.concatenate(
        [np.concatenate([32 * k + 2 * np.arange(16),
                         32 * k + 2 * np.arange(16) + 1])
         for k in range(D // 32)]
    )
    wa_t = We1[:, :D].T
    wb_t = We1[:, D:2 * D].T
    wc_t = We1[:, 2 * D:].T[:, pi]
    u = (We2.T @ Wc1[0])[pi]
    k1 = be2 @ Wc1[0] + bc1[0]
    consts = jnp.zeros((16,), jnp.float32)
    consts = consts.at[0].set(k1).at[1].set(Wc2[0, 0]).at[2].set(bc2[0])
    idx4 = jnp.stack(
        [edge_index[0].reshape(_NW, NCH, C), edge_index[1].reshape(_NW, NCH, C)],
        axis=2,
    )

    ta, tb = pl.pallas_call(
        _tc_pre_body,
        out_shape=[
            jax.ShapeDtypeStruct((N, D), jnp.bfloat16),
            jax.ShapeDtypeStruct((N, D), jnp.bfloat16),
        ],
    )(x, wa_t, wb_t)

    BE = 4000
    a_tab = pl.pallas_call(
        _tc_attr_body,
        grid=(E // BE,),
        in_specs=[
            pl.BlockSpec((BE, edge_attr.shape[1]), lambda i: (i, 0)),
            pl.BlockSpec(wc_t.shape, lambda i: (0, 0)),
            pl.BlockSpec((1, D), lambda i: (0, 0)),
        ],
        out_specs=pl.BlockSpec((BE, D), lambda i: (i, 0)),
        out_shape=jax.ShapeDtypeStruct((E, D), jnp.float32),
    )(edge_attr, wc_t, be1[pi][None, :])

    hp, sp = _make_sc_edge_kernel(N, E, D, C)(ta, tb, a_tab, idx4, u, consts)

    nf, pos_out = pl.pallas_call(
        _tc_post_body,
        out_shape=[
            jax.ShapeDtypeStruct((N, D), jnp.float32),
            jax.ShapeDtypeStruct(pos.shape, jnp.float32),
        ],
    )(x, hp, sp, pos,
      We2.T[pi, :], be2[None, :], Wn1[:, :D].T, Wn1[:, D:].T, bn1[None, :],
      Wn2.T, bn2[None, :])
    return (nf, pos_out)
